# baseline (speedup 1.0000x reference)
import functools
import jax
import jax.numpy as jnp
from jax import lax
from jax.experimental import pallas as pl
from jax.experimental.pallas import tpu as pltpu
from jax.experimental.pallas import tpu_sc as plsc

THRESH = 0.1
EPS = 1e-8
NC = 2
NS = 16
NW = NC * NS
LANES = 16
W = 80

_mesh = functools.partial(
    plsc.VectorSubcoreMesh, core_axis_name="c", subcore_axis_name="s",
    num_cores=NC, num_subcores=NS)

_SC_PARAMS = pltpu.CompilerParams(use_tc_tiling_on_sc=False,
                                  needs_layout_passes=False)


def _wid():
    cid = lax.axis_index("c")
    sid = lax.axis_index("s")
    return cid, sid, sid * NC + cid


def _iota16():
    return lax.iota(jnp.int32, LANES)


def _splat(v):
    return jnp.full((LANES,), v, jnp.int32)


def _ring(nwin, nb, start_fn, wait_fn, work_fn):
    for b in range(nb):
        start_fn(b, b)
    nblocks = nwin // nb

    @pl.loop(0, nblocks)
    def _t(t):
        for b in range(nb):
            w = t * nb + b
            wait_fn(w, b)
            work_fn(w, b)
            nxt = w + nb

            @pl.when(nxt < nwin)
            def _s():
                start_fn(nxt, b)

    for b in range(nwin - nblocks * nb):
        w = nblocks * nb + b
        wait_fn(w, b)
        work_fn(w, b)


def _sc_sim128(xn, rc2, row2, zeros16):
    N, D = xn.shape
    NWIN = row2.shape[0] // NW
    W2X = 2 * W
    RPS = N // NS
    NB = 3

    @functools.partial(
        pl.kernel,
        out_type=(jax.ShapeDtypeStruct(row2.shape, jnp.float32),
                  jax.ShapeDtypeStruct((NC, N, LANES), jnp.float32)),
        mesh=_mesh(),
        compiler_params=_SC_PARAMS,
        scratch_types=[
            pltpu.VMEM((NWIN, W2X), jnp.int32),
            pltpu.VMEM((NWIN, W), jnp.int32),
            pltpu.VMEM((NB, W2X, D), jnp.float32),
            pltpu.VMEM((W, LANES), jnp.float32),
            pltpu.VMEM((NWIN, W), jnp.float32),
            pltpu.VMEM_SHARED((N, LANES), jnp.float32),
        ] + [pltpu.SemaphoreType.DMA] * NB,
    )
    def k(xn_h, rc_h, row_h, z16_h, att_h, tab_h,
          rcb, idxr, abuf, payload, attw, tab_s, *sems):
        cid, sid, wid = _wid()
        pltpu.sync_copy(z16_h.at[pl.ds(sid * RPS, RPS)],
                        tab_s.at[pl.ds(sid * RPS, RPS)])
        pltpu.sync_copy(z16_h.at[pl.ds(0, W)], payload)
        pltpu.sync_copy(rc_h.at[pl.ds(wid * NWIN, NWIN)], rcb)
        pltpu.sync_copy(row_h.at[pl.ds(wid * NWIN, NWIN)], idxr)

        plsc.subcore_barrier()

        def start(w, b):
            pltpu.async_copy(xn_h.at[rcb.at[w]], abuf.at[b], sems[b])

        def wait(w, b):
            pltpu.make_async_copy(xn_h.at[rcb.at[w]], abuf.at[b],
                                  sems[b]).wait()

        def work(w, b):
            for g in range(W // LANES):
                g16 = _iota16() + (g * LANES)
                h16 = g16 + W

                iot = _iota16()

                def dot16(kb, accs):
                    accs = list(accs)
                    for u in range(16):
                        ks = (jnp.full((LANES,), kb * 16 + u, jnp.int32)
                              + iot) & (D - 1)
                        va = plsc.load_gather(abuf.at[b], [g16, ks])
                        vb = plsc.load_gather(abuf.at[b], [h16, ks])
                        accs[u % 4] = accs[u % 4] + va * vb
                    return tuple(accs)

                z = jnp.zeros((LANES,), jnp.float32)
                a0, a1, a2, a3 = lax.fori_loop(0, D // 16, dot16,
                                               (z, z, z, z))
                sim = (a0 + a1) + (a2 + a3)
                att = jnp.where(sim < THRESH, 0.0, sim)
                nz = jnp.where(att > 0.0, 1.0, 0.0)
                plsc.store_scatter(attw, [_splat(w), g16], att)
                plsc.store_scatter(payload, [g16, _splat(0)], att)
                plsc.store_scatter(payload, [g16, _splat(1)], nz)
            pltpu.sync_copy(payload, tab_s.at[idxr.at[w]], add=True)

        _ring(NWIN, NB, start, wait, work)

        pltpu.sync_copy(attw, att_h.at[pl.ds(wid * NWIN, NWIN)])
        plsc.subcore_barrier()
        pltpu.sync_copy(tab_s.at[pl.ds(sid * RPS, RPS)],
                        tab_h.at[cid, pl.ds(sid * RPS, RPS)])

    return k(xn, rc2, row2, zeros16)


def _sc_ew_scatter(nodetab, att2, row2, col2, zeros16):
    N = nodetab.shape[0]
    NWIN = row2.shape[0] // NW
    RPS = N // NS
    NB = 4

    @functools.partial(
        pl.kernel,
        out_type=jax.ShapeDtypeStruct((NC, N, LANES), jnp.float32),
        mesh=_mesh(),
        compiler_params=_SC_PARAMS,
        scratch_types=[
            pltpu.VMEM((NWIN, W), jnp.int32),
            pltpu.VMEM((NWIN, W), jnp.int32),
            pltpu.VMEM((NWIN, W), jnp.float32),
            pltpu.VMEM((NB, W, LANES), jnp.float32),
            pltpu.VMEM((W, LANES), jnp.float32),
            pltpu.VMEM_SHARED((N, LANES), jnp.float32),
        ] + [pltpu.SemaphoreType.DMA] * NB,
    )
    def k(nt_h, att_h, row_h, col_h, z16_h, tab_h,
          idxr, idxc, attb, nbuf, payload, tab_s, *sems):
        cid, sid, wid = _wid()
        pltpu.sync_copy(z16_h.at[pl.ds(sid * RPS, RPS)],
                        tab_s.at[pl.ds(sid * RPS, RPS)])
        pltpu.sync_copy(z16_h.at[pl.ds(0, W)], payload)
        pltpu.sync_copy(row_h.at[pl.ds(wid * NWIN, NWIN)], idxr)
        pltpu.sync_copy(col_h.at[pl.ds(wid * NWIN, NWIN)], idxc)
        pltpu.sync_copy(att_h.at[pl.ds(wid * NWIN, NWIN)], attb)

        plsc.subcore_barrier()

        def start(w, b):
            pltpu.async_copy(nt_h.at[idxr.at[w]], nbuf.at[b], sems[b])

        def wait(w, b):
            pltpu.make_async_copy(nt_h.at[idxr.at[w]], nbuf.at[b],
                                  sems[b]).wait()

        def work(w, b):
            for g in range(W // LANES):
                g16 = _iota16() + (g * LANES)
                iv = plsc.load_gather(nbuf.at[b], [g16, _splat(0)])
                attv = plsc.load_gather(attb, [_splat(w), g16])
                plsc.store_scatter(payload, [g16, _splat(0)], attv * iv)
            pltpu.sync_copy(payload, tab_s.at[idxc.at[w]], add=True)

        _ring(NWIN, NB, start, wait, work)

        plsc.subcore_barrier()
        pltpu.sync_copy(tab_s.at[pl.ds(sid * RPS, RPS)],
                        tab_h.at[cid, pl.ds(sid * RPS, RPS)])

    return k(nodetab, att2, row2, col2, zeros16)


def _sc_aggregate(hnt, att2, rc2, col2, zeros16):
    N = hnt.shape[0]
    D = LANES
    NWIN = col2.shape[0] // NW
    W2X = 2 * W
    RPS = N // NS
    NB = 4

    @functools.partial(
        pl.kernel,
        out_type=jax.ShapeDtypeStruct((NC, N, D), jnp.float32),
        mesh=_mesh(),
        compiler_params=_SC_PARAMS,
        scratch_types=[
            pltpu.VMEM((NWIN, W2X), jnp.int32),
            pltpu.VMEM((NWIN, W), jnp.int32),
            pltpu.VMEM((NWIN, W), jnp.float32),
            pltpu.VMEM((NB, W2X, 2 * LANES), jnp.float32),
            pltpu.VMEM((W, D), jnp.float32),
            pltpu.VMEM_SHARED((N, D), jnp.float32),
        ] + [pltpu.SemaphoreType.DMA] * NB,
    )
    def k(hnt_h, att_h, rc_h, col_h, z16_h, tab_h,
          rcb, idxc, attb, gbuf, payload, tab_s, *sems):
        cid, sid, wid = _wid()
        pltpu.sync_copy(z16_h.at[pl.ds(sid * RPS, RPS)],
                        tab_s.at[pl.ds(sid * RPS, RPS)])
        pltpu.sync_copy(rc_h.at[pl.ds(wid * NWIN, NWIN)], rcb)
        pltpu.sync_copy(col_h.at[pl.ds(wid * NWIN, NWIN)], idxc)
        pltpu.sync_copy(att_h.at[pl.ds(wid * NWIN, NWIN)], attb)

        plsc.subcore_barrier()

        def start(w, b):
            pltpu.async_copy(hnt_h.at[rcb.at[w]], gbuf.at[b], sems[b])

        def wait(w, b):
            pltpu.make_async_copy(hnt_h.at[rcb.at[w]], gbuf.at[b],
                                  sems[b]).wait()

        def work(w, b):
            for g in range(W // LANES):
                g16 = _iota16() + (g * LANES)
                wr = plsc.load_gather(gbuf.at[b], [g16, _splat(17)])
                wc = plsc.load_gather(gbuf.at[b], [g16 + W, _splat(16)])
                attv = plsc.load_gather(attb, [_splat(w), g16])
                wv = wr * attv * wc
                iot = _iota16()
                for jj in range(D):
                    js = (_splat(jj) + iot) & (D - 1)
                    hv = plsc.load_gather(gbuf.at[b], [g16, js])
                    plsc.store_scatter(payload, [g16, js], wv * hv)
            pltpu.sync_copy(payload, tab_s.at[idxc.at[w]], add=True)

        _ring(NWIN, NB, start, wait, work)

        plsc.subcore_barrier()
        pltpu.sync_copy(tab_s.at[pl.ds(sid * RPS, RPS)],
                        tab_h.at[cid, pl.ds(sid * RPS, RPS)])

    return k(hnt, att2, rc2, col2, zeros16)


def _sc_sim16(haug, rc2, row2, zeros16):
    N, D2 = haug.shape
    NWIN = row2.shape[0] // NW
    W2X = 2 * W
    RPS = N // NS
    NB = 4

    @functools.partial(
        pl.kernel,
        out_type=(jax.ShapeDtypeStruct(row2.shape, jnp.float32),
                  jax.ShapeDtypeStruct((NC, N, LANES), jnp.float32)),
        mesh=_mesh(),
        compiler_params=_SC_PARAMS,
        scratch_types=[
            pltpu.VMEM((NWIN, W2X), jnp.int32),
            pltpu.VMEM((NWIN, W), jnp.int32),
            pltpu.VMEM((NB, W2X, D2), jnp.float32),
            pltpu.VMEM((W, LANES), jnp.float32),
            pltpu.VMEM((NWIN, W), jnp.float32),
            pltpu.VMEM_SHARED((N, LANES), jnp.float32),
        ] + [pltpu.SemaphoreType.DMA] * NB,
    )
    def k(h_h, rc_h, row_h, z16_h, att_h, tab_h,
          rcb, idxr, abuf, payload, attw, tab_s, *sems):
        cid, sid, wid = _wid()
        pltpu.sync_copy(z16_h.at[pl.ds(sid * RPS, RPS)],
                        tab_s.at[pl.ds(sid * RPS, RPS)])
        pltpu.sync_copy(z16_h.at[pl.ds(0, W)], payload)
        pltpu.sync_copy(rc_h.at[pl.ds(wid * NWIN, NWIN)], rcb)
        pltpu.sync_copy(row_h.at[pl.ds(wid * NWIN, NWIN)], idxr)

        plsc.subcore_barrier()

        def start(w, b):
            pltpu.async_copy(h_h.at[rcb.at[w]], abuf.at[b], sems[b])

        def wait(w, b):
            pltpu.make_async_copy(h_h.at[rcb.at[w]], abuf.at[b],
                                  sems[b]).wait()

        def work(w, b):
            for g in range(W // LANES):
                g16 = _iota16() + (g * LANES)
                h16 = g16 + W
                z = jnp.zeros((LANES,), jnp.float32)
                accs = [z, z, z, z]
                iot = _iota16()
                for kk in range(LANES):
                    ks = (_splat(kk) + iot) & (LANES - 1)
                    va = plsc.load_gather(abuf.at[b], [g16, ks])
                    vb = plsc.load_gather(abuf.at[b], [h16, ks])
                    accs[kk % 4] = accs[kk % 4] + va * vb
                acc = (accs[0] + accs[1]) + (accs[2] + accs[3])
                na = plsc.load_gather(abuf.at[b], [g16, _splat(LANES)])
                nb_ = plsc.load_gather(abuf.at[b], [h16, _splat(LANES)])
                sim = acc / (na * nb_ + EPS)
                att = jnp.where(sim < THRESH, 0.0, sim)
                plsc.store_scatter(attw, [_splat(w), g16], att)
                plsc.store_scatter(payload, [g16, _splat(0)], att)
            pltpu.sync_copy(payload, tab_s.at[idxr.at[w]], add=True)

        _ring(NWIN, NB, start, wait, work)

        pltpu.sync_copy(attw, att_h.at[pl.ds(wid * NWIN, NWIN)])
        plsc.subcore_barrier()
        pltpu.sync_copy(tab_s.at[pl.ds(sid * RPS, RPS)],
                        tab_h.at[cid, pl.ds(sid * RPS, RPS)])

    return k(haug, rc2, row2, zeros16)


def _tc(body, out_shapes, *args):
    return pl.pallas_call(
        body, out_shape=out_shapes,
        compiler_params=pltpu.CompilerParams(
            vmem_limit_bytes=64 * 1024 * 1024))(*args)


def _p1_prep(x, W1):
    N, D = x.shape
    H = W1.shape[1]

    def body(x_r, w1_r, xn_r, hpre_r):
        xx = x_r[...]
        na = jnp.sqrt(jnp.sum(xx * xx, axis=1, keepdims=True))
        xn_r[...] = jnp.where(na > 0.0, xx / na, 0.0)
        hpre_r[...] = jnp.dot(xx, w1_r[...],
                              preferred_element_type=jnp.float32,
                              precision=lax.Precision.HIGHEST)

    return _tc(body, (jax.ShapeDtypeStruct((N, D), jnp.float32),
                      jax.ShapeDtypeStruct((N, H), jnp.float32)), x, W1)


def _p2_nodetab_a(tab1):
    N = tab1.shape[1]

    def body(t_r, o_r):
        t = t_r[0] + t_r[1]
        rs = t[:, 0:1]
        cnt = t[:, 1:2]
        invrs = jnp.where(rs > 0.0, 1.0 / rs, 0.0)
        selfw = 1.0 / (cnt + 1.0)
        o_r[...] = jnp.concatenate(
            [invrs, selfw, jnp.zeros((N, LANES - 2), jnp.float32)], axis=1)

    return _tc(body, jax.ShapeDtypeStruct((N, LANES), jnp.float32), tab1)


def _p3_nodetab_b(tab2, nta, hpre):
    N, H = hpre.shape

    def body(t_r, a_r, hp_r, o_r):
        deg = t_r[0][:, 0:1] + t_r[1][:, 0:1] + a_r[:, 1:2]
        dinv = jnp.where(deg > 0.0, lax.rsqrt(deg), 0.0)
        o_r[...] = jnp.concatenate(
            [hp_r[...], dinv, dinv * a_r[:, 0:1],
             jnp.zeros((N, LANES - 2), jnp.float32)], axis=1)

    return _tc(body, jax.ShapeDtypeStruct((N, H + LANES), jnp.float32),
               tab2, nta, hpre)


def _p4_hidden(tab3, hpre, nta, ntb, b1):
    N, H = hpre.shape

    def body(t_r, hp_r, a_r, b_r, b1_r, haug_r, h1_r, a2s_r):
        hp = hp_r[...]
        selfw = a_r[:, 1:2]
        dinv = b_r[:, 16:17]
        h1 = t_r[0] + t_r[1] + selfw * dinv * dinv * hp + b1_r[...][None, :]
        h1 = jnp.maximum(h1, 0.0)
        q = jnp.sqrt(jnp.sum(h1 * h1, axis=1, keepdims=True))
        haug_r[...] = jnp.concatenate(
            [h1, jnp.broadcast_to(q, (N, H))], axis=1)
        h1_r[...] = h1
        q2 = q * q
        sim = q2 / (q2 + EPS)
        a2s_r[...] = jnp.where(sim < THRESH, 0.0, sim)

    return _tc(body, (jax.ShapeDtypeStruct((N, 2 * H), jnp.float32),
                      jax.ShapeDtypeStruct((N, H), jnp.float32),
                      jax.ShapeDtypeStruct((N, 1), jnp.float32)),
               tab3, hpre, nta, ntb, b1)


def _p5_nodetab_c(tab4, a2s):
    N = tab4.shape[1]

    def body(t_r, s_r, o_r):
        rs = t_r[0][:, 0:1] + t_r[1][:, 0:1] + s_r[...]
        invrs = jnp.where(rs > 0.0, 1.0 / rs, 0.0)
        o_r[...] = jnp.concatenate(
            [invrs, jnp.zeros((N, LANES - 1), jnp.float32)], axis=1)

    return _tc(body, jax.ShapeDtypeStruct((N, LANES), jnp.float32), tab4, a2s)


def _p6_nodetab_d(tab5, ntc, a2s, h1):
    N, H = h1.shape

    def body(t_r, c_r, s_r, h1_r, o_r):
        invrs = c_r[:, 0:1]
        deg = t_r[0][:, 0:1] + t_r[1][:, 0:1] + s_r[...] * invrs
        dinv = jnp.where(deg > 0.0, lax.rsqrt(deg), 0.0)
        o_r[...] = jnp.concatenate(
            [h1_r[...], dinv, dinv * invrs,
             jnp.zeros((N, LANES - 2), jnp.float32)], axis=1)

    return _tc(body, jax.ShapeDtypeStruct((N, H + LANES), jnp.float32),
               tab5, ntc, a2s, h1)


def _p7_out(tab6, ntc, ntd, a2s, h1, W2, b2):
    N = h1.shape[0]
    DO = W2.shape[1]

    def body(t_r, c_r, d_r, s_r, h1_r, w2_r, b2_r, o_r):
        invrs = c_r[:, 0:1]
        dinv = d_r[:, 16:17]
        pre = t_r[0] + t_r[1] + dinv * dinv * s_r[...] * invrs * h1_r[...]
        o_r[...] = jnp.dot(pre, w2_r[...],
                           preferred_element_type=jnp.float32,
                           precision=lax.Precision.HIGHEST) + b2_r[...][None, :]

    return _tc(body, jax.ShapeDtypeStruct((N, DO), jnp.float32),
               tab6, ntc, ntd, a2s, h1, W2, b2)


@jax.jit
def kernel(x, edge_index, W1, b1, W2, b2):
    N, D = x.shape
    E = edge_index.shape[1]
    assert E % (NW * W) == 0 and N % NS == 0

    row2 = edge_index[0].astype(jnp.int32).reshape(E // W, W)
    col2 = edge_index[1].astype(jnp.int32).reshape(E // W, W)
    rc2 = jnp.concatenate([row2, col2], axis=1)
    zeros16 = jnp.zeros((N, LANES), jnp.float32)

    xn, hpre = _p1_prep(x, W1)
    att, tab1 = _sc_sim128(xn, rc2, row2, zeros16)
    nta = _p2_nodetab_a(tab1)
    tab2 = _sc_ew_scatter(nta, att, row2, col2, zeros16)
    ntb = _p3_nodetab_b(tab2, nta, hpre)
    tab3 = _sc_aggregate(ntb, att, rc2, col2, zeros16)
    haug, h1, a2s = _p4_hidden(tab3, hpre, nta, ntb, b1)
    att2, tab4 = _sc_sim16(haug, rc2, row2, zeros16)
    ntc = _p5_nodetab_c(tab4, a2s)
    tab5 = _sc_ew_scatter(ntc, att2, row2, col2, zeros16)
    ntd = _p6_nodetab_d(tab5, ntc, a2s, h1)
    tab6 = _sc_aggregate(ntd, att2, rc2, col2, zeros16)
    out = _p7_out(tab6, ntc, ntd, a2s, h1, W2, b2)
    return out


# speedup vs baseline: 1.0889x; 1.0889x over previous
import functools
import jax
import jax.numpy as jnp
from jax import lax
from jax.experimental import pallas as pl
from jax.experimental.pallas import tpu as pltpu
from jax.experimental.pallas import tpu_sc as plsc

THRESH = 0.1
EPS = 1e-8
NC = 2
NS = 16
NW = NC * NS
LANES = 16
W = 80

_mesh = functools.partial(
    plsc.VectorSubcoreMesh, core_axis_name="c", subcore_axis_name="s",
    num_cores=NC, num_subcores=NS)

_SC_PARAMS = pltpu.CompilerParams(use_tc_tiling_on_sc=False,
                                  needs_layout_passes=False)


def _wid():
    cid = lax.axis_index("c")
    sid = lax.axis_index("s")
    return cid, sid, sid * NC + cid


def _iota16():
    return lax.iota(jnp.int32, LANES)


def _splat(v):
    return jnp.full((LANES,), v, jnp.int32)


def _ring(nwin, nb, start_fn, wait_fn, work_fn):
    for b in range(nb):
        start_fn(b, b)
    nblocks = nwin // nb

    @pl.loop(0, nblocks)
    def _t(t):
        for b in range(nb):
            w = t * nb + b
            wait_fn(w, b)
            work_fn(w, b)
            nxt = w + nb

            @pl.when(nxt < nwin)
            def _s():
                start_fn(nxt, b)

    for b in range(nwin - nblocks * nb):
        w = nblocks * nb + b
        wait_fn(w, b)
        work_fn(w, b)


def _sc_sim128(xn, rc2, row2, zeros16):
    N, D = xn.shape
    NWIN = row2.shape[0] // NW
    W2X = 2 * W
    RPS = N // NS
    NB = 3

    @functools.partial(
        pl.kernel,
        out_type=(jax.ShapeDtypeStruct(row2.shape, jnp.float32),
                  jax.ShapeDtypeStruct((NC, N, LANES), jnp.float32)),
        mesh=_mesh(),
        compiler_params=_SC_PARAMS,
        scratch_types=[
            pltpu.VMEM((NWIN, W2X), jnp.int32),
            pltpu.VMEM((NWIN, W), jnp.int32),
            pltpu.VMEM((NB, W2X, D), jnp.float32),
            pltpu.VMEM((W, LANES), jnp.float32),
            pltpu.VMEM((NWIN, W), jnp.float32),
            pltpu.VMEM_SHARED((N, LANES), jnp.float32),
        ] + [pltpu.SemaphoreType.DMA] * NB,
    )
    def k(xn_h, rc_h, row_h, z16_h, att_h, tab_h,
          rcb, idxr, abuf, payload, attw, tab_s, *sems):
        cid, sid, wid = _wid()
        pltpu.sync_copy(z16_h.at[pl.ds(sid * RPS, RPS)],
                        tab_s.at[pl.ds(sid * RPS, RPS)])
        pltpu.sync_copy(z16_h.at[pl.ds(0, W)], payload)
        pltpu.sync_copy(rc_h.at[pl.ds(wid * NWIN, NWIN)], rcb)
        pltpu.sync_copy(row_h.at[pl.ds(wid * NWIN, NWIN)], idxr)

        plsc.subcore_barrier()

        def start(w, b):
            pltpu.async_copy(xn_h.at[rcb.at[w]], abuf.at[b], sems[b])

        def wait(w, b):
            pltpu.make_async_copy(xn_h.at[rcb.at[w]], abuf.at[b],
                                  sems[b]).wait()

        def work(w, b):
            for g in range(W // LANES):
                g16 = _iota16() + (g * LANES)
                h16 = g16 + W

                iot = _iota16()

                def dot16(kb, accs):
                    accs = list(accs)
                    for u in range(16):
                        ks = (jnp.full((LANES,), kb * 16 + u, jnp.int32)
                              + iot) & (D - 1)
                        va = plsc.load_gather(abuf.at[b], [g16, ks])
                        vb = plsc.load_gather(abuf.at[b], [h16, ks])
                        accs[u % 4] = accs[u % 4] + va * vb
                    return tuple(accs)

                z = jnp.zeros((LANES,), jnp.float32)
                a0, a1, a2, a3 = lax.fori_loop(0, D // 16, dot16,
                                               (z, z, z, z))
                sim = (a0 + a1) + (a2 + a3)
                att = jnp.where(sim < THRESH, 0.0, sim)
                nz = jnp.where(att > 0.0, 1.0, 0.0)
                plsc.store_scatter(attw, [_splat(w), g16], att)
                plsc.store_scatter(payload, [g16, _splat(0)], att)
                plsc.store_scatter(payload, [g16, _splat(1)], nz)
            pltpu.sync_copy(payload, tab_s.at[idxr.at[w]], add=True)

        _ring(NWIN, NB, start, wait, work)

        pltpu.sync_copy(attw, att_h.at[pl.ds(wid * NWIN, NWIN)])
        plsc.subcore_barrier()
        pltpu.sync_copy(tab_s.at[pl.ds(sid * RPS, RPS)],
                        tab_h.at[cid, pl.ds(sid * RPS, RPS)])

    return k(xn, rc2, row2, zeros16)


def _sc_ew_scatter(nodetab, att2, row2, col2, zeros16):
    N = nodetab.shape[0]
    NWIN = row2.shape[0] // NW
    RPS = N // NS
    NB = 4

    @functools.partial(
        pl.kernel,
        out_type=jax.ShapeDtypeStruct((NC, N, LANES), jnp.float32),
        mesh=_mesh(),
        compiler_params=_SC_PARAMS,
        scratch_types=[
            pltpu.VMEM((NWIN, W), jnp.int32),
            pltpu.VMEM((NWIN, W), jnp.int32),
            pltpu.VMEM((NWIN, W), jnp.float32),
            pltpu.VMEM((NB, W, LANES), jnp.float32),
            pltpu.VMEM((W, LANES), jnp.float32),
            pltpu.VMEM_SHARED((N, LANES), jnp.float32),
        ] + [pltpu.SemaphoreType.DMA] * NB,
    )
    def k(nt_h, att_h, row_h, col_h, z16_h, tab_h,
          idxr, idxc, attb, nbuf, payload, tab_s, *sems):
        cid, sid, wid = _wid()
        pltpu.sync_copy(z16_h.at[pl.ds(sid * RPS, RPS)],
                        tab_s.at[pl.ds(sid * RPS, RPS)])
        pltpu.sync_copy(z16_h.at[pl.ds(0, W)], payload)
        pltpu.sync_copy(row_h.at[pl.ds(wid * NWIN, NWIN)], idxr)
        pltpu.sync_copy(col_h.at[pl.ds(wid * NWIN, NWIN)], idxc)
        pltpu.sync_copy(att_h.at[pl.ds(wid * NWIN, NWIN)], attb)

        plsc.subcore_barrier()

        def start(w, b):
            pltpu.async_copy(nt_h.at[idxr.at[w]], nbuf.at[b], sems[b])

        def wait(w, b):
            pltpu.make_async_copy(nt_h.at[idxr.at[w]], nbuf.at[b],
                                  sems[b]).wait()

        def work(w, b):
            for g in range(W // LANES):
                g16 = _iota16() + (g * LANES)
                iv = plsc.load_gather(nbuf.at[b], [g16, _splat(0)])
                attv = plsc.load_gather(attb, [_splat(w), g16])
                plsc.store_scatter(payload, [g16, _splat(0)], attv * iv)
            pltpu.sync_copy(payload, tab_s.at[idxc.at[w]], add=True)

        _ring(NWIN, NB, start, wait, work)

        plsc.subcore_barrier()
        pltpu.sync_copy(tab_s.at[pl.ds(sid * RPS, RPS)],
                        tab_h.at[cid, pl.ds(sid * RPS, RPS)])

    return k(nodetab, att2, row2, col2, zeros16)


def _sc_aggregate(h, nodetab, att2, rc2, col2, zeros16):
    N, D = h.shape
    NWIN = col2.shape[0] // NW
    W2X = 2 * W
    RPS = N // NS
    NB = 4

    @functools.partial(
        pl.kernel,
        out_type=jax.ShapeDtypeStruct((NC, N, D), jnp.float32),
        mesh=_mesh(),
        compiler_params=_SC_PARAMS,
        scratch_types=[
            pltpu.VMEM((NWIN, W2X), jnp.int32),
            pltpu.VMEM((NWIN, W), jnp.int32),
            pltpu.VMEM((NWIN, W), jnp.float32),
            pltpu.VMEM((NB, W, D), jnp.float32),
            pltpu.VMEM((NB, W2X, LANES), jnp.float32),
            pltpu.VMEM_SHARED((N, D), jnp.float32),
        ] + [pltpu.SemaphoreType.DMA] * NB,
    )
    def k(h_h, nt_h, att_h, rc_h, col_h, z16_h, tab_h,
          rcb, idxc, attb, hbuf, nbuf, tab_s, *sems):
        cid, sid, wid = _wid()
        pltpu.sync_copy(z16_h.at[pl.ds(sid * RPS, RPS)],
                        tab_s.at[pl.ds(sid * RPS, RPS)])
        pltpu.sync_copy(rc_h.at[pl.ds(wid * NWIN, NWIN)], rcb)
        pltpu.sync_copy(col_h.at[pl.ds(wid * NWIN, NWIN)], idxc)
        pltpu.sync_copy(att_h.at[pl.ds(wid * NWIN, NWIN)], attb)

        plsc.subcore_barrier()

        def start(w, b):
            pltpu.async_copy(h_h.at[rcb.at[w].at[pl.ds(0, W)]],
                             hbuf.at[b], sems[b])
            pltpu.async_copy(nt_h.at[rcb.at[w]], nbuf.at[b], sems[b])

        def wait(w, b):
            pltpu.make_async_copy(h_h.at[rcb.at[w].at[pl.ds(0, W)]],
                                  hbuf.at[b], sems[b]).wait()
            pltpu.make_async_copy(nt_h.at[rcb.at[w]], nbuf.at[b],
                                  sems[b]).wait()

        def work(w, b):
            for g in range(W // LANES):
                g16 = _iota16() + (g * LANES)
                wr = plsc.load_gather(nbuf.at[b], [g16, _splat(1)])
                wc = plsc.load_gather(nbuf.at[b], [g16 + W, _splat(0)])
                attv = plsc.load_gather(attb, [_splat(w), g16])
                wv = wr * attv * wc
                iot = _iota16()
                for jj in range(D):
                    js = (_splat(jj) + iot) & (D - 1)
                    hv = plsc.load_gather(hbuf.at[b], [g16, js])
                    plsc.store_scatter(hbuf.at[b], [g16, js], wv * hv)
            pltpu.sync_copy(hbuf.at[b], tab_s.at[idxc.at[w]], add=True)

        _ring(NWIN, NB, start, wait, work)

        plsc.subcore_barrier()
        pltpu.sync_copy(tab_s.at[pl.ds(sid * RPS, RPS)],
                        tab_h.at[cid, pl.ds(sid * RPS, RPS)])

    return k(h, nodetab, att2, rc2, col2, zeros16)


def _sc_sim16(haug, rc2, row2, zeros16):
    N, D2 = haug.shape
    NWIN = row2.shape[0] // NW
    W2X = 2 * W
    RPS = N // NS
    NB = 4

    @functools.partial(
        pl.kernel,
        out_type=(jax.ShapeDtypeStruct(row2.shape, jnp.float32),
                  jax.ShapeDtypeStruct((NC, N, LANES), jnp.float32)),
        mesh=_mesh(),
        compiler_params=_SC_PARAMS,
        scratch_types=[
            pltpu.VMEM((NWIN, W2X), jnp.int32),
            pltpu.VMEM((NWIN, W), jnp.int32),
            pltpu.VMEM((NB, W2X, D2), jnp.float32),
            pltpu.VMEM((W, LANES), jnp.float32),
            pltpu.VMEM((NWIN, W), jnp.float32),
            pltpu.VMEM_SHARED((N, LANES), jnp.float32),
        ] + [pltpu.SemaphoreType.DMA] * NB,
    )
    def k(h_h, rc_h, row_h, z16_h, att_h, tab_h,
          rcb, idxr, abuf, payload, attw, tab_s, *sems):
        cid, sid, wid = _wid()
        pltpu.sync_copy(z16_h.at[pl.ds(sid * RPS, RPS)],
                        tab_s.at[pl.ds(sid * RPS, RPS)])
        pltpu.sync_copy(z16_h.at[pl.ds(0, W)], payload)
        pltpu.sync_copy(rc_h.at[pl.ds(wid * NWIN, NWIN)], rcb)
        pltpu.sync_copy(row_h.at[pl.ds(wid * NWIN, NWIN)], idxr)

        plsc.subcore_barrier()

        def start(w, b):
            pltpu.async_copy(h_h.at[rcb.at[w]], abuf.at[b], sems[b])

        def wait(w, b):
            pltpu.make_async_copy(h_h.at[rcb.at[w]], abuf.at[b],
                                  sems[b]).wait()

        def work(w, b):
            for g in range(W // LANES):
                g16 = _iota16() + (g * LANES)
                h16 = g16 + W
                z = jnp.zeros((LANES,), jnp.float32)
                accs = [z, z, z, z]
                iot = _iota16()
                for kk in range(LANES):
                    ks = (_splat(kk) + iot) & (LANES - 1)
                    va = plsc.load_gather(abuf.at[b], [g16, ks])
                    vb = plsc.load_gather(abuf.at[b], [h16, ks])
                    accs[kk % 4] = accs[kk % 4] + va * vb
                acc = (accs[0] + accs[1]) + (accs[2] + accs[3])
                na = plsc.load_gather(abuf.at[b], [g16, _splat(LANES)])
                nb_ = plsc.load_gather(abuf.at[b], [h16, _splat(LANES)])
                sim = acc / (na * nb_ + EPS)
                att = jnp.where(sim < THRESH, 0.0, sim)
                plsc.store_scatter(attw, [_splat(w), g16], att)
                plsc.store_scatter(payload, [g16, _splat(0)], att)
            pltpu.sync_copy(payload, tab_s.at[idxr.at[w]], add=True)

        _ring(NWIN, NB, start, wait, work)

        pltpu.sync_copy(attw, att_h.at[pl.ds(wid * NWIN, NWIN)])
        plsc.subcore_barrier()
        pltpu.sync_copy(tab_s.at[pl.ds(sid * RPS, RPS)],
                        tab_h.at[cid, pl.ds(sid * RPS, RPS)])

    return k(haug, rc2, row2, zeros16)


def _tc(body, out_shapes, *args):
    return pl.pallas_call(
        body, out_shape=out_shapes,
        compiler_params=pltpu.CompilerParams(
            vmem_limit_bytes=64 * 1024 * 1024))(*args)


def _p1_prep(x, W1):
    N, D = x.shape
    H = W1.shape[1]

    def body(x_r, w1_r, xn_r, hpre_r):
        xx = x_r[...]
        na = jnp.sqrt(jnp.sum(xx * xx, axis=1, keepdims=True))
        xn_r[...] = jnp.where(na > 0.0, xx / na, 0.0)
        hpre_r[...] = jnp.dot(xx, w1_r[...],
                              preferred_element_type=jnp.float32,
                              precision=lax.Precision.HIGHEST)

    return _tc(body, (jax.ShapeDtypeStruct((N, D), jnp.float32),
                      jax.ShapeDtypeStruct((N, H), jnp.float32)), x, W1)


def _p2_nodetab_a(tab1):
    N = tab1.shape[1]

    def body(t_r, o_r):
        t = t_r[0] + t_r[1]
        rs = t[:, 0:1]
        cnt = t[:, 1:2]
        invrs = jnp.where(rs > 0.0, 1.0 / rs, 0.0)
        selfw = 1.0 / (cnt + 1.0)
        o_r[...] = jnp.concatenate(
            [invrs, selfw, jnp.zeros((N, LANES - 2), jnp.float32)], axis=1)

    return _tc(body, jax.ShapeDtypeStruct((N, LANES), jnp.float32), tab1)


def _p3_nodetab_b(tab2, nta):
    N = tab2.shape[1]

    def body(t_r, a_r, o_r):
        deg = t_r[0][:, 0:1] + t_r[1][:, 0:1] + a_r[:, 1:2]
        dinv = jnp.where(deg > 0.0, lax.rsqrt(deg), 0.0)
        o_r[...] = jnp.concatenate(
            [dinv, dinv * a_r[:, 0:1],
             jnp.zeros((N, LANES - 2), jnp.float32)], axis=1)

    return _tc(body, jax.ShapeDtypeStruct((N, LANES), jnp.float32), tab2, nta)


def _p4_hidden(tab3, hpre, nta, ntb, b1):
    N, H = hpre.shape

    def body(t_r, hp_r, a_r, b_r, b1_r, haug_r, h1_r, a2s_r):
        hp = hp_r[...]
        selfw = a_r[:, 1:2]
        dinv = b_r[:, 0:1]
        h1 = t_r[0] + t_r[1] + selfw * dinv * dinv * hp + b1_r[...][None, :]
        h1 = jnp.maximum(h1, 0.0)
        q = jnp.sqrt(jnp.sum(h1 * h1, axis=1, keepdims=True))
        haug_r[...] = jnp.concatenate(
            [h1, jnp.broadcast_to(q, (N, H))], axis=1)
        h1_r[...] = h1
        q2 = q * q
        sim = q2 / (q2 + EPS)
        a2s_r[...] = jnp.where(sim < THRESH, 0.0, sim)

    return _tc(body, (jax.ShapeDtypeStruct((N, 2 * H), jnp.float32),
                      jax.ShapeDtypeStruct((N, H), jnp.float32),
                      jax.ShapeDtypeStruct((N, 1), jnp.float32)),
               tab3, hpre, nta, ntb, b1)


def _p5_nodetab_c(tab4, a2s):
    N = tab4.shape[1]

    def body(t_r, s_r, o_r):
        rs = t_r[0][:, 0:1] + t_r[1][:, 0:1] + s_r[...]
        invrs = jnp.where(rs > 0.0, 1.0 / rs, 0.0)
        o_r[...] = jnp.concatenate(
            [invrs, jnp.zeros((N, LANES - 1), jnp.float32)], axis=1)

    return _tc(body, jax.ShapeDtypeStruct((N, LANES), jnp.float32), tab4, a2s)


def _p6_nodetab_d(tab5, ntc, a2s):
    N = tab5.shape[1]

    def body(t_r, c_r, s_r, o_r):
        invrs = c_r[:, 0:1]
        deg = t_r[0][:, 0:1] + t_r[1][:, 0:1] + s_r[...] * invrs
        dinv = jnp.where(deg > 0.0, lax.rsqrt(deg), 0.0)
        o_r[...] = jnp.concatenate(
            [dinv, dinv * invrs, jnp.zeros((N, LANES - 2), jnp.float32)],
            axis=1)

    return _tc(body, jax.ShapeDtypeStruct((N, LANES), jnp.float32),
               tab5, ntc, a2s)


def _p7_out(tab6, ntc, ntd, a2s, h1, W2, b2):
    N = h1.shape[0]
    DO = W2.shape[1]

    def body(t_r, c_r, d_r, s_r, h1_r, w2_r, b2_r, o_r):
        invrs = c_r[:, 0:1]
        dinv = d_r[:, 0:1]
        pre = t_r[0] + t_r[1] + dinv * dinv * s_r[...] * invrs * h1_r[...]
        o_r[...] = jnp.dot(pre, w2_r[...],
                           preferred_element_type=jnp.float32,
                           precision=lax.Precision.HIGHEST) + b2_r[...][None, :]

    return _tc(body, jax.ShapeDtypeStruct((N, DO), jnp.float32),
               tab6, ntc, ntd, a2s, h1, W2, b2)


@jax.jit
def kernel(x, edge_index, W1, b1, W2, b2):
    N, D = x.shape
    E = edge_index.shape[1]
    assert E % (NW * W) == 0 and N % NS == 0

    row2 = edge_index[0].astype(jnp.int32).reshape(E // W, W)
    col2 = edge_index[1].astype(jnp.int32).reshape(E // W, W)
    rc2 = jnp.concatenate([row2, col2], axis=1)
    zeros16 = jnp.zeros((N, LANES), jnp.float32)

    xn, hpre = _p1_prep(x, W1)
    att, tab1 = _sc_sim128(xn, rc2, row2, zeros16)
    nta = _p2_nodetab_a(tab1)
    tab2 = _sc_ew_scatter(nta, att, row2, col2, zeros16)
    ntb = _p3_nodetab_b(tab2, nta)
    tab3 = _sc_aggregate(hpre, ntb, att, rc2, col2, zeros16)
    haug, h1, a2s = _p4_hidden(tab3, hpre, nta, ntb, b1)
    att2, tab4 = _sc_sim16(haug, rc2, row2, zeros16)
    ntc = _p5_nodetab_c(tab4, a2s)
    tab5 = _sc_ew_scatter(ntc, att2, row2, col2, zeros16)
    ntd = _p6_nodetab_d(tab5, ntc, a2s)
    tab6 = _sc_aggregate(h1, ntd, att2, rc2, col2, zeros16)
    out = _p7_out(tab6, ntc, ntd, a2s, h1, W2, b2)
    return out


# speedup vs baseline: 1.2220x; 1.1222x over previous
import functools
import jax
import jax.numpy as jnp
from jax import lax
from jax.experimental import pallas as pl
from jax.experimental.pallas import tpu as pltpu
from jax.experimental.pallas import tpu_sc as plsc

THRESH = 0.1
EPS = 1e-8
NC = 2
NS = 16
NW = NC * NS
LANES = 16
W = 80

_mesh = functools.partial(
    plsc.VectorSubcoreMesh, core_axis_name="c", subcore_axis_name="s",
    num_cores=NC, num_subcores=NS)

_SC_PARAMS = pltpu.CompilerParams(use_tc_tiling_on_sc=False,
                                  needs_layout_passes=False)


def _wid():
    cid = lax.axis_index("c")
    sid = lax.axis_index("s")
    return cid, sid, sid * NC + cid


def _iota16():
    return lax.iota(jnp.int32, LANES)


def _splat(v):
    return jnp.full((LANES,), v, jnp.int32)


def _ring(nwin, nb, start_fn, wait_fn, work_fn):
    for b in range(nb):
        start_fn(b, b)
    nblocks = nwin // nb

    @pl.loop(0, nblocks)
    def _t(t):
        for b in range(nb):
            w = t * nb + b
            wait_fn(w, b)
            work_fn(w, b)
            nxt = w + nb

            @pl.when(nxt < nwin)
            def _s():
                start_fn(nxt, b)

    for b in range(nwin - nblocks * nb):
        w = nblocks * nb + b
        wait_fn(w, b)
        work_fn(w, b)


def _sc_sim128(xn, rc2, row2, zeros16):
    N, D = xn.shape
    NWIN = row2.shape[0] // NW
    W2X = 2 * W
    RPS = N // NS
    NB = 3

    @functools.partial(
        pl.kernel,
        out_type=(jax.ShapeDtypeStruct(row2.shape, jnp.float32),
                  jax.ShapeDtypeStruct((NC, N, LANES), jnp.float32)),
        mesh=_mesh(),
        compiler_params=_SC_PARAMS,
        scratch_types=[
            pltpu.VMEM((NWIN, W2X), jnp.int32),
            pltpu.VMEM((NWIN, W), jnp.int32),
            pltpu.VMEM((NB, W2X, D), jnp.float32),
            pltpu.VMEM((W, LANES), jnp.float32),
            pltpu.VMEM((NWIN, W), jnp.float32),
            pltpu.VMEM_SHARED((N, LANES), jnp.float32),
        ] + [pltpu.SemaphoreType.DMA] * NB,
    )
    def k(xn_h, rc_h, row_h, z16_h, att_h, tab_h,
          rcb, idxr, abuf, payload, attw, tab_s, *sems):
        cid, sid, wid = _wid()
        pltpu.sync_copy(z16_h.at[pl.ds(sid * RPS, RPS)],
                        tab_s.at[pl.ds(sid * RPS, RPS)])
        pltpu.sync_copy(z16_h.at[pl.ds(0, W)], payload)
        pltpu.sync_copy(rc_h.at[pl.ds(wid * NWIN, NWIN)], rcb)
        pltpu.sync_copy(row_h.at[pl.ds(wid * NWIN, NWIN)], idxr)

        plsc.subcore_barrier()

        def start(w, b):
            pltpu.async_copy(xn_h.at[rcb.at[w]], abuf.at[b], sems[b])

        def wait(w, b):
            pltpu.make_async_copy(xn_h.at[rcb.at[w]], abuf.at[b],
                                  sems[b]).wait()

        def work(w, b):
            for g in range(W // LANES):
                g16 = _iota16() + (g * LANES)
                h16 = g16 + W

                iot = _iota16()

                def dot16(kb, accs):
                    accs = list(accs)
                    for u in range(16):
                        ks = (jnp.full((LANES,), kb * 16 + u, jnp.int32)
                              + iot) & (D - 1)
                        va = plsc.load_gather(abuf.at[b], [g16, ks])
                        vb = plsc.load_gather(abuf.at[b], [h16, ks])
                        accs[u % 4] = accs[u % 4] + va * vb
                    return tuple(accs)

                z = jnp.zeros((LANES,), jnp.float32)
                a0, a1, a2, a3 = lax.fori_loop(0, D // 16, dot16,
                                               (z, z, z, z))
                sim = (a0 + a1) + (a2 + a3)
                att = jnp.where(sim < THRESH, 0.0, sim)
                nz = jnp.where(att > 0.0, 1.0, 0.0)
                plsc.store_scatter(attw, [_splat(w), g16], att)
                plsc.store_scatter(payload, [g16, _splat(0)], att)
                plsc.store_scatter(payload, [g16, _splat(1)], nz)
            pltpu.sync_copy(payload, tab_s.at[idxr.at[w]], add=True)

        _ring(NWIN, NB, start, wait, work)

        pltpu.sync_copy(attw, att_h.at[pl.ds(wid * NWIN, NWIN)])
        plsc.subcore_barrier()
        pltpu.sync_copy(tab_s.at[pl.ds(sid * RPS, RPS)],
                        tab_h.at[cid, pl.ds(sid * RPS, RPS)])

    return k(xn, rc2, row2, zeros16)


def _sc_ew_scatter(nodetab, att2, row2, col2, zeros16):
    N = nodetab.shape[0]
    W = row2.shape[1]
    NWIN = row2.shape[0] // NW
    RPS = N // NS
    NB = 4

    @functools.partial(
        pl.kernel,
        out_type=jax.ShapeDtypeStruct((NC, N, LANES), jnp.float32),
        mesh=_mesh(),
        compiler_params=_SC_PARAMS,
        scratch_types=[
            pltpu.VMEM((NWIN, W), jnp.int32),
            pltpu.VMEM((NWIN, W), jnp.int32),
            pltpu.VMEM((NWIN, W), jnp.float32),
            pltpu.VMEM((NB, W, LANES), jnp.float32),
            pltpu.VMEM((W, LANES), jnp.float32),
            pltpu.VMEM_SHARED((N, LANES), jnp.float32),
        ] + [pltpu.SemaphoreType.DMA] * NB,
    )
    def k(nt_h, att_h, row_h, col_h, z16_h, tab_h,
          idxr, idxc, attb, nbuf, payload, tab_s, *sems):
        cid, sid, wid = _wid()
        pltpu.sync_copy(z16_h.at[pl.ds(sid * RPS, RPS)],
                        tab_s.at[pl.ds(sid * RPS, RPS)])
        pltpu.sync_copy(z16_h.at[pl.ds(0, W)], payload)
        pltpu.sync_copy(row_h.at[pl.ds(wid * NWIN, NWIN)], idxr)
        pltpu.sync_copy(col_h.at[pl.ds(wid * NWIN, NWIN)], idxc)
        pltpu.sync_copy(att_h.at[pl.ds(wid * NWIN, NWIN)], attb)

        plsc.subcore_barrier()

        def start(w, b):
            pltpu.async_copy(nt_h.at[idxr.at[w]], nbuf.at[b], sems[b])

        def wait(w, b):
            pltpu.make_async_copy(nt_h.at[idxr.at[w]], nbuf.at[b],
                                  sems[b]).wait()

        def work(w, b):
            @pl.loop(0, W // LANES)
            def _g(g):
                g16 = _iota16() + g * LANES
                iv = plsc.load_gather(nbuf.at[b], [g16, _splat(0)])
                attv = plsc.load_gather(attb, [_splat(w), g16])
                plsc.store_scatter(payload, [g16, _splat(0)], attv * iv)
            pltpu.sync_copy(payload, tab_s.at[idxc.at[w]], add=True)

        _ring(NWIN, NB, start, wait, work)

        plsc.subcore_barrier()
        pltpu.sync_copy(tab_s.at[pl.ds(sid * RPS, RPS)],
                        tab_h.at[cid, pl.ds(sid * RPS, RPS)])

    return k(nodetab, att2, row2, col2, zeros16)


def _sc_aggregate(h, nodetab, att2, rc2, col2, zeros16):
    N, D = h.shape
    W = col2.shape[1]
    NWIN = col2.shape[0] // NW
    W2X = 2 * W
    RPS = N // NS
    NB = 4

    @functools.partial(
        pl.kernel,
        out_type=jax.ShapeDtypeStruct((NC, N, D), jnp.float32),
        mesh=_mesh(),
        compiler_params=_SC_PARAMS,
        scratch_types=[
            pltpu.VMEM((NWIN, W2X), jnp.int32),
            pltpu.VMEM((NWIN, W), jnp.int32),
            pltpu.VMEM((NWIN, W), jnp.float32),
            pltpu.VMEM((NB, W, D), jnp.float32),
            pltpu.VMEM((NB, W2X, LANES), jnp.float32),
            pltpu.VMEM_SHARED((N, D), jnp.float32),
        ] + [pltpu.SemaphoreType.DMA] * NB,
    )
    def k(h_h, nt_h, att_h, rc_h, col_h, z16_h, tab_h,
          rcb, idxc, attb, hbuf, nbuf, tab_s, *sems):
        cid, sid, wid = _wid()
        pltpu.sync_copy(z16_h.at[pl.ds(sid * RPS, RPS)],
                        tab_s.at[pl.ds(sid * RPS, RPS)])
        pltpu.sync_copy(rc_h.at[pl.ds(wid * NWIN, NWIN)], rcb)
        pltpu.sync_copy(col_h.at[pl.ds(wid * NWIN, NWIN)], idxc)
        pltpu.sync_copy(att_h.at[pl.ds(wid * NWIN, NWIN)], attb)

        plsc.subcore_barrier()

        def start(w, b):
            pltpu.async_copy(h_h.at[rcb.at[w].at[pl.ds(0, W)]],
                             hbuf.at[b], sems[b])
            pltpu.async_copy(nt_h.at[rcb.at[w]], nbuf.at[b], sems[b])

        def wait(w, b):
            pltpu.make_async_copy(h_h.at[rcb.at[w].at[pl.ds(0, W)]],
                                  hbuf.at[b], sems[b]).wait()
            pltpu.make_async_copy(nt_h.at[rcb.at[w]], nbuf.at[b],
                                  sems[b]).wait()

        def work(w, b):
            @pl.loop(0, W // LANES)
            def _g(g):
                g16 = _iota16() + g * LANES
                wr = plsc.load_gather(nbuf.at[b], [g16, _splat(1)])
                wc = plsc.load_gather(nbuf.at[b], [g16 + W, _splat(0)])
                attv = plsc.load_gather(attb, [_splat(w), g16])
                wv = wr * attv * wc
                iot = _iota16()
                for jj in range(D):
                    js = (_splat(jj) + iot) & (D - 1)
                    hv = plsc.load_gather(hbuf.at[b], [g16, js])
                    plsc.store_scatter(hbuf.at[b], [g16, js], wv * hv)
            pltpu.sync_copy(hbuf.at[b], tab_s.at[idxc.at[w]], add=True)

        _ring(NWIN, NB, start, wait, work)

        plsc.subcore_barrier()
        pltpu.sync_copy(tab_s.at[pl.ds(sid * RPS, RPS)],
                        tab_h.at[cid, pl.ds(sid * RPS, RPS)])

    return k(h, nodetab, att2, rc2, col2, zeros16)


def _sc_sim16(haug, rc2, row2, zeros16):
    N, D2 = haug.shape
    W = row2.shape[1]
    NWIN = row2.shape[0] // NW
    W2X = 2 * W
    RPS = N // NS
    NB = 2

    @functools.partial(
        pl.kernel,
        out_type=(jax.ShapeDtypeStruct(row2.shape, jnp.float32),
                  jax.ShapeDtypeStruct((NC, N, LANES), jnp.float32)),
        mesh=_mesh(),
        compiler_params=_SC_PARAMS,
        scratch_types=[
            pltpu.VMEM((NWIN, W2X), jnp.int32),
            pltpu.VMEM((NWIN, W), jnp.int32),
            pltpu.VMEM((NB, W2X, D2), jnp.float32),
            pltpu.VMEM((W, LANES), jnp.float32),
            pltpu.VMEM((NWIN, W), jnp.float32),
            pltpu.VMEM_SHARED((N, LANES), jnp.float32),
        ] + [pltpu.SemaphoreType.DMA] * NB,
    )
    def k(h_h, rc_h, row_h, z16_h, att_h, tab_h,
          rcb, idxr, abuf, payload, attw, tab_s, *sems):
        cid, sid, wid = _wid()
        pltpu.sync_copy(z16_h.at[pl.ds(sid * RPS, RPS)],
                        tab_s.at[pl.ds(sid * RPS, RPS)])
        pltpu.sync_copy(z16_h.at[pl.ds(0, W)], payload)
        pltpu.sync_copy(rc_h.at[pl.ds(wid * NWIN, NWIN)], rcb)
        pltpu.sync_copy(row_h.at[pl.ds(wid * NWIN, NWIN)], idxr)

        plsc.subcore_barrier()

        def start(w, b):
            pltpu.async_copy(h_h.at[rcb.at[w]], abuf.at[b], sems[b])

        def wait(w, b):
            pltpu.make_async_copy(h_h.at[rcb.at[w]], abuf.at[b],
                                  sems[b]).wait()

        def work(w, b):
            @pl.loop(0, W // LANES)
            def _g(g):
                g16 = _iota16() + g * LANES
                h16 = g16 + W
                z = jnp.zeros((LANES,), jnp.float32)
                accs = [z, z, z, z]
                iot = _iota16()
                for kk in range(LANES):
                    ks = (_splat(kk) + iot) & (LANES - 1)
                    va = plsc.load_gather(abuf.at[b], [g16, ks])
                    vb = plsc.load_gather(abuf.at[b], [h16, ks])
                    accs[kk % 4] = accs[kk % 4] + va * vb
                acc = (accs[0] + accs[1]) + (accs[2] + accs[3])
                na = plsc.load_gather(abuf.at[b], [g16, _splat(LANES)])
                nb_ = plsc.load_gather(abuf.at[b], [h16, _splat(LANES)])
                sim = acc / (na * nb_ + EPS)
                att = jnp.where(sim < THRESH, 0.0, sim)
                plsc.store_scatter(attw, [_splat(w), g16], att)
                plsc.store_scatter(payload, [g16, _splat(0)], att)
            pltpu.sync_copy(payload, tab_s.at[idxr.at[w]], add=True)

        _ring(NWIN, NB, start, wait, work)

        pltpu.sync_copy(attw, att_h.at[pl.ds(wid * NWIN, NWIN)])
        plsc.subcore_barrier()
        pltpu.sync_copy(tab_s.at[pl.ds(sid * RPS, RPS)],
                        tab_h.at[cid, pl.ds(sid * RPS, RPS)])

    return k(haug, rc2, row2, zeros16)


def _tc(body, out_shapes, *args):
    return pl.pallas_call(
        body, out_shape=out_shapes,
        compiler_params=pltpu.CompilerParams(
            vmem_limit_bytes=64 * 1024 * 1024))(*args)


def _p1_prep(x, W1):
    N, D = x.shape
    H = W1.shape[1]

    def body(x_r, w1_r, xn_r, hpre_r):
        xx = x_r[...]
        na = jnp.sqrt(jnp.sum(xx * xx, axis=1, keepdims=True))
        xn_r[...] = jnp.where(na > 0.0, xx / na, 0.0)
        hpre_r[...] = jnp.dot(xx, w1_r[...],
                              preferred_element_type=jnp.float32,
                              precision=lax.Precision.HIGHEST)

    return _tc(body, (jax.ShapeDtypeStruct((N, D), jnp.float32),
                      jax.ShapeDtypeStruct((N, H), jnp.float32)), x, W1)


def _p2_nodetab_a(tab1):
    N = tab1.shape[1]

    def body(t_r, o_r):
        t = t_r[0] + t_r[1]
        rs = t[:, 0:1]
        cnt = t[:, 1:2]
        invrs = jnp.where(rs > 0.0, 1.0 / rs, 0.0)
        selfw = 1.0 / (cnt + 1.0)
        o_r[...] = jnp.concatenate(
            [invrs, selfw, jnp.zeros((N, LANES - 2), jnp.float32)], axis=1)

    return _tc(body, jax.ShapeDtypeStruct((N, LANES), jnp.float32), tab1)


def _p3_nodetab_b(tab2, nta):
    N = tab2.shape[1]

    def body(t_r, a_r, o_r):
        deg = t_r[0][:, 0:1] + t_r[1][:, 0:1] + a_r[:, 1:2]
        dinv = jnp.where(deg > 0.0, lax.rsqrt(deg), 0.0)
        o_r[...] = jnp.concatenate(
            [dinv, dinv * a_r[:, 0:1],
             jnp.zeros((N, LANES - 2), jnp.float32)], axis=1)

    return _tc(body, jax.ShapeDtypeStruct((N, LANES), jnp.float32), tab2, nta)


def _p4_hidden(tab3, hpre, nta, ntb, b1):
    N, H = hpre.shape

    def body(t_r, hp_r, a_r, b_r, b1_r, haug_r, h1_r, a2s_r):
        hp = hp_r[...]
        selfw = a_r[:, 1:2]
        dinv = b_r[:, 0:1]
        h1 = t_r[0] + t_r[1] + selfw * dinv * dinv * hp + b1_r[...][None, :]
        h1 = jnp.maximum(h1, 0.0)
        q = jnp.sqrt(jnp.sum(h1 * h1, axis=1, keepdims=True))
        haug_r[...] = jnp.concatenate(
            [h1, jnp.broadcast_to(q, (N, H))], axis=1)
        h1_r[...] = h1
        q2 = q * q
        sim = q2 / (q2 + EPS)
        a2s_r[...] = jnp.where(sim < THRESH, 0.0, sim)

    return _tc(body, (jax.ShapeDtypeStruct((N, 2 * H), jnp.float32),
                      jax.ShapeDtypeStruct((N, H), jnp.float32),
                      jax.ShapeDtypeStruct((N, 1), jnp.float32)),
               tab3, hpre, nta, ntb, b1)


def _p5_nodetab_c(tab4, a2s):
    N = tab4.shape[1]

    def body(t_r, s_r, o_r):
        rs = t_r[0][:, 0:1] + t_r[1][:, 0:1] + s_r[...]
        invrs = jnp.where(rs > 0.0, 1.0 / rs, 0.0)
        o_r[...] = jnp.concatenate(
            [invrs, jnp.zeros((N, LANES - 1), jnp.float32)], axis=1)

    return _tc(body, jax.ShapeDtypeStruct((N, LANES), jnp.float32), tab4, a2s)


def _p6_nodetab_d(tab5, ntc, a2s):
    N = tab5.shape[1]

    def body(t_r, c_r, s_r, o_r):
        invrs = c_r[:, 0:1]
        deg = t_r[0][:, 0:1] + t_r[1][:, 0:1] + s_r[...] * invrs
        dinv = jnp.where(deg > 0.0, lax.rsqrt(deg), 0.0)
        o_r[...] = jnp.concatenate(
            [dinv, dinv * invrs, jnp.zeros((N, LANES - 2), jnp.float32)],
            axis=1)

    return _tc(body, jax.ShapeDtypeStruct((N, LANES), jnp.float32),
               tab5, ntc, a2s)


def _p7_out(tab6, ntc, ntd, a2s, h1, W2, b2):
    N = h1.shape[0]
    DO = W2.shape[1]

    def body(t_r, c_r, d_r, s_r, h1_r, w2_r, b2_r, o_r):
        invrs = c_r[:, 0:1]
        dinv = d_r[:, 0:1]
        pre = t_r[0] + t_r[1] + dinv * dinv * s_r[...] * invrs * h1_r[...]
        o_r[...] = jnp.dot(pre, w2_r[...],
                           preferred_element_type=jnp.float32,
                           precision=lax.Precision.HIGHEST) + b2_r[...][None, :]

    return _tc(body, jax.ShapeDtypeStruct((N, DO), jnp.float32),
               tab6, ntc, ntd, a2s, h1, W2, b2)


@jax.jit
def kernel(x, edge_index, W1, b1, W2, b2):
    N, D = x.shape
    E = edge_index.shape[1]
    assert E % (NW * W) == 0 and N % NS == 0

    WB = 400
    row2 = edge_index[0].astype(jnp.int32).reshape(E // W, W)
    col2 = edge_index[1].astype(jnp.int32).reshape(E // W, W)
    rc2 = jnp.concatenate([row2, col2], axis=1)
    row2b = edge_index[0].astype(jnp.int32).reshape(E // WB, WB)
    col2b = edge_index[1].astype(jnp.int32).reshape(E // WB, WB)
    rc2b = jnp.concatenate([row2b, col2b], axis=1)
    zeros16 = jnp.zeros((N, LANES), jnp.float32)

    xn, hpre = _p1_prep(x, W1)
    att, tab1 = _sc_sim128(xn, rc2, row2, zeros16)
    attb = att.reshape(E // WB, WB)
    nta = _p2_nodetab_a(tab1)
    tab2 = _sc_ew_scatter(nta, attb, row2b, col2b, zeros16)
    ntb = _p3_nodetab_b(tab2, nta)
    tab3 = _sc_aggregate(hpre, ntb, attb, rc2b, col2b, zeros16)
    haug, h1, a2s = _p4_hidden(tab3, hpre, nta, ntb, b1)
    att2, tab4 = _sc_sim16(haug, rc2b, row2b, zeros16)
    ntc = _p5_nodetab_c(tab4, a2s)
    tab5 = _sc_ew_scatter(ntc, att2, row2b, col2b, zeros16)
    ntd = _p6_nodetab_d(tab5, ntc, a2s)
    tab6 = _sc_aggregate(h1, ntd, att2, rc2b, col2b, zeros16)
    out = _p7_out(tab6, ntc, ntd, a2s, h1, W2, b2)
    return out
